# Initial kernel scaffold; baseline (speedup 1.0000x reference)
import functools

import jax
import jax.numpy as jnp
from jax import lax
from jax.experimental import pallas as pl
from jax.experimental.pallas import tpu as pltpu
from jax.experimental.pallas import tpu_sc as plsc

BATCH = 16384
IDS_PER_ROW = 10
D = 64
B = BATCH * IDS_PER_ROW
NW = 32
B_PER_W = B // NW
CHUNK = 1024
NCHUNK = B_PER_W // CHUNK


def _emb_body(champ_tab, team_tab, cids, tids, out, idx_v, tid_v, rows_v,
              team_v, sem):
    c = lax.axis_index("c")
    s = lax.axis_index("s")
    wid = s * 2 + c
    base = wid * B_PER_W

    pltpu.sync_copy(team_tab, team_v)
    pltpu.sync_copy(cids.at[pl.ds(base, B_PER_W)], idx_v)
    pltpu.sync_copy(tids.at[pl.ds(base, B_PER_W)], tid_v)

    t0 = [team_v[0, pl.ds(g * 16, 16)] for g in range(4)]
    t1 = [team_v[1, pl.ds(g * 16, 16)] for g in range(4)]

    for ck in range(NCHUNK):
        off = ck * CHUNK
        pltpu.async_copy(
            champ_tab.at[idx_v.at[pl.ds(off, CHUNK)]], rows_v, sem
        ).wait()

        def row_body(r, carry):
            rsplat = jnp.full((16,), off, jnp.int32) + r
            tv = plsc.load_gather(tid_v, [rsplat])
            m = tv > 0
            for g in range(4):
                add = jnp.where(m, t1[g], t0[g])
                rows_v[r, pl.ds(g * 16, 16)] = (
                    rows_v[r, pl.ds(g * 16, 16)] + add
                )
            return carry

        lax.fori_loop(0, CHUNK, row_body, 0)
        pltpu.sync_copy(rows_v, out.at[pl.ds(base + off, CHUNK)])


@jax.jit
def _emb_call(champ_tab, team_tab, cids, tids):
    kern = pl.kernel(
        _emb_body,
        out_type=jax.ShapeDtypeStruct((B, D), jnp.float32),
        mesh=plsc.VectorSubcoreMesh(core_axis_name="c", subcore_axis_name="s"),
        scratch_types=[
            pltpu.VMEM((B_PER_W,), jnp.int32),
            pltpu.VMEM((B_PER_W,), jnp.int32),
            pltpu.VMEM((CHUNK, D), jnp.float32),
            pltpu.VMEM((2, D), jnp.float32),
            pltpu.SemaphoreType.DMA,
        ],
    )
    return kern(champ_tab, team_tab, cids, tids)


def kernel(numeric_features, champ_ids, team_ids, role_ids, subclass_ids,
           scaling_ids, champion_table, team_table):
    cids = champ_ids.reshape(-1).astype(jnp.int32)
    tids = team_ids.reshape(-1).astype(jnp.int32)
    flat = _emb_call(champion_table, team_table, cids, tids)
    return flat.reshape(BATCH, IDS_PER_ROW, D)


# speedup vs baseline: 1.2586x; 1.2586x over previous
import functools

import jax
import jax.numpy as jnp
from jax import lax
from jax.experimental import pallas as pl
from jax.experimental.pallas import tpu as pltpu
from jax.experimental.pallas import tpu_sc as plsc

BATCH = 16384
IDS_PER_ROW = 10
D = 64
B = BATCH * IDS_PER_ROW
NW = 32
B_PER_W = B // NW
CHUNK = 1024
NCHUNK = B_PER_W // CHUNK


def _emb_body(champ_tab, team_tab, cids, tids, out, idx_v, tid_v, rows_v,
              team_v, sem):
    c = lax.axis_index("c")
    s = lax.axis_index("s")
    wid = s * 2 + c
    base = wid * B_PER_W

    pltpu.sync_copy(team_tab, team_v)
    pltpu.sync_copy(cids.at[pl.ds(base, B_PER_W)], idx_v)
    pltpu.sync_copy(tids.at[pl.ds(base, B_PER_W)], tid_v)

    t0 = [team_v[0, pl.ds(g * 16, 16)] for g in range(4)]
    t1 = [team_v[1, pl.ds(g * 16, 16)] for g in range(4)]

    for ck in range(NCHUNK):
        off = ck * CHUNK
        pltpu.async_copy(
            champ_tab.at[idx_v.at[pl.ds(off, CHUNK)]], rows_v, sem
        ).wait()

        def row_body(r, carry):
            rsplat = jnp.full((16,), off, jnp.int32) + r
            tv = plsc.load_gather(tid_v, [rsplat])
            m = tv > 0
            for g in range(4):
                add = jnp.where(m, t1[g], t0[g])
                rows_v[r, pl.ds(g * 16, 16)] = (
                    rows_v[r, pl.ds(g * 16, 16)] + add
                )
            return carry

        lax.fori_loop(0, CHUNK, row_body, 0)
        pltpu.sync_copy(rows_v, out.at[pl.ds(base + off, CHUNK)])


@jax.jit
def _emb_call(champ_tab, team_tab, cids, tids):
    kern = pl.kernel(
        _emb_body,
        out_type=jax.ShapeDtypeStruct((B, D), jnp.float32),
        mesh=plsc.VectorSubcoreMesh(core_axis_name="c", subcore_axis_name="s"),
        scratch_types=[
            pltpu.VMEM((B_PER_W,), jnp.int32),
            pltpu.VMEM((B_PER_W,), jnp.int32),
            pltpu.VMEM((CHUNK, D), jnp.float32),
            pltpu.VMEM((2, D), jnp.float32),
            pltpu.SemaphoreType.DMA,
        ],
        compiler_params=pltpu.CompilerParams(
            needs_layout_passes=False, use_tc_tiling_on_sc=False
        ),
    )
    return kern(champ_tab, team_tab, cids, tids)


def kernel(numeric_features, champ_ids, team_ids, role_ids, subclass_ids,
           scaling_ids, champion_table, team_table):
    cids = champ_ids.reshape(-1).astype(jnp.int32)
    tids = team_ids.reshape(-1).astype(jnp.int32)
    flat = _emb_call(champion_table, team_table, cids, tids)
    return flat.reshape(BATCH, IDS_PER_ROW, D)
